# baseline (speedup 1.0000x reference)
import jax
import jax.numpy as jnp
from jax.experimental import pallas as pl
from jax.experimental.pallas import tpu as pltpu

_E = 8
_K = 2
_H = 768
_BT = 2048
_NS = 4
_BS = _BT // _NS


def _router_body(x0, x1, x2, x3, w_ref, b_ref, logits_ref, aff_ref, idx_ref):
    w = w_ref[...]
    b = b_ref[...]
    for j, xr in enumerate((x0, x1, x2, x3)):
        lg = jax.lax.dot_general(
            xr[...], w, (((1,), (1,)), ((), ())),
            preferred_element_type=jnp.float32) + b
        logits_ref[pl.ds(j * _BS, _BS), :] = lg
        aff_ref[pl.ds(j * _BS, _BS), :] = lg
    idx_ref[...] = jnp.zeros_like(idx_ref)


@jax.jit
def kernel(hidden_states, W, b):
    x = hidden_states.reshape(-1, _H)
    t = x.shape[0]
    b2 = b.reshape(1, _E)

    def xspec(j):
        return pl.BlockSpec((_BS, _H), lambda i, j=j: (i * _NS + j, 0))

    logits, aff, idx = pl.pallas_call(
        _router_body,
        grid=(t // _BT,),
        in_specs=[xspec(0), xspec(1), xspec(2), xspec(3),
                  pl.BlockSpec((_E, _H), lambda i: (0, 0)),
                  pl.BlockSpec((1, _E), lambda i: (0, 0))],
        out_specs=[
            pl.BlockSpec((_BT, _E), lambda i: (i, 0)),
            pl.BlockSpec((_BT, _E), lambda i: (i, 0)),
            pl.BlockSpec((_BT, _K), lambda i: (i, 0)),
        ],
        out_shape=[
            jax.ShapeDtypeStruct((t, _E), jnp.float32),
            jax.ShapeDtypeStruct((t, _E), jnp.float32),
            jax.ShapeDtypeStruct((t, _K), jnp.int32),
        ],
        compiler_params=pltpu.CompilerParams(
            dimension_semantics=("arbitrary",)),
    )(x, x, x, x, W, b2)
    return (logits, aff, idx)
